# Initial kernel scaffold; baseline (speedup 1.0000x reference)
import jax
import jax.numpy as jnp
from jax.experimental import pallas as pl


def kernel(x, edge_index, batch, W0, b0, gnw0, gnb0, gnm0, W1, b1, gnw1, gnb1, gnm1, W2, b2, gnw2, gnb2, gnm2, Wd1, bd1, Wd2, bd2):
    raise NotImplementedError("write your pallas kernel here")


# speedup vs baseline: 10.2915x; 10.2915x over previous
import functools

import jax
import jax.numpy as jnp
from jax import lax
from jax.experimental import pallas as pl
from jax.experimental.pallas import tpu as pltpu
from jax.experimental.pallas import tpu_sc as plsc

N = 10000
NPAD = 10240
E = 320000
D = 128
H = 128
G = 64
C = 32

NW = 16
K = 128
CH = 160
CHC = CH // 2
EW = CH * K
EPAD = NW * EW
RPT = NPAD // 16

BLK = 1024
GRID = NPAD // BLK

_HI = lax.Precision.HIGHEST


def _mesh():
    return plsc.VectorSubcoreMesh(core_axis_name="c", subcore_axis_name="s",
                                  num_cores=2, num_subcores=16)


_SC_PARAMS = pltpu.CompilerParams(use_tc_tiling_on_sc=False)


@functools.cache
def _build_sc_deg():
    return functools.partial(
        pl.kernel,
        out_type=jax.ShapeDtypeStruct((2, NPAD, 16), jnp.float32),
        mesh=_mesh(),
        scratch_types=[
            pltpu.VMEM((CH, K), jnp.int32),
            pltpu.VMEM((K, 16), jnp.float32),
            pltpu.VMEM_SHARED((NPAD, 16), jnp.float32),
        ],
        compiler_params=_SC_PARAMS,
    )(_sc_deg_body)


def _sc_deg(dstw, ones16, zeros16):
    return _build_sc_deg()(dstw, ones16, zeros16)


def _sc_deg_body(dst_hbm, ones_hbm, z_hbm, out_hbm, idx_v, ones_v, acc):
    core = lax.axis_index("c")
    sub = lax.axis_index("s")

    @pl.when(sub == 0)
    def _():
        pltpu.sync_copy(z_hbm, acc)

    plsc.subcore_barrier()
    pltpu.sync_copy(dst_hbm.at[sub], idx_v)
    pltpu.sync_copy(ones_hbm, ones_v)
    off = core * CHC

    def body(j, carry):
        pltpu.sync_copy(ones_v, acc.at[idx_v.at[off + j]], add=True)
        return carry

    lax.fori_loop(0, CHC, body, 0)
    plsc.subcore_barrier()
    start = sub * RPT
    pltpu.sync_copy(acc.at[pl.ds(start, RPT)], out_hbm.at[core].at[pl.ds(start, RPT)])


@functools.cache
def _build_sc_scatter():
    return functools.partial(
        pl.kernel,
        out_type=jax.ShapeDtypeStruct((2, NPAD, 64), jnp.float32),
        mesh=_mesh(),
        scratch_types=[
            pltpu.VMEM((CH, K), jnp.int32),
            pltpu.VMEM((CH, K), jnp.int32),
            pltpu.VMEM((K, 64), jnp.float32),
            pltpu.VMEM((K, 64), jnp.float32),
            pltpu.VMEM_SHARED((NPAD, 64), jnp.float32),
            pltpu.SemaphoreType.DMA,
            pltpu.SemaphoreType.DMA,
        ],
        compiler_params=_SC_PARAMS,
    )(_sc_scatter_body)


def _sc_scatter(hs2, srcw, dstw, zeros64):
    return _build_sc_scatter()(hs2, srcw, dstw, zeros64)


def _sc_scatter_body(hs_hbm, src_hbm, dst_hbm, z_hbm, out_hbm,
                     sidx, didx, rows0, rows1, acc, sem0, sem1):
    core = lax.axis_index("c")
    sub = lax.axis_index("s")

    @pl.when(sub == 0)
    def _():
        pltpu.sync_copy(z_hbm, acc)

    plsc.subcore_barrier()
    pltpu.sync_copy(src_hbm.at[sub], sidx)
    pltpu.sync_copy(dst_hbm.at[sub], didx)
    table = hs_hbm.at[core]

    def _drain(buf, sem):
        pltpu.make_async_copy(z_hbm.at[pl.ds(0, K)], buf, sem).wait()

    pltpu.async_copy(table.at[sidx.at[0]], rows0, sem0)

    def body(j, carry):
        c0 = 2 * j
        pltpu.async_copy(table.at[sidx.at[c0 + 1]], rows1, sem1)
        _drain(rows0, sem0)
        pltpu.sync_copy(rows0, acc.at[didx.at[c0]], add=True)

        @pl.when(c0 + 2 < CH)
        def _():
            pltpu.async_copy(table.at[sidx.at[c0 + 2]], rows0, sem0)

        _drain(rows1, sem1)
        pltpu.sync_copy(rows1, acc.at[didx.at[c0 + 1]], add=True)
        return carry

    lax.fori_loop(0, CH // 2, body, 0)
    plsc.subcore_barrier()
    start = sub * RPT
    pltpu.sync_copy(acc.at[pl.ds(start, RPT)], out_hbm.at[core].at[pl.ds(start, RPT)])


def _k0_body(x_r, w_r, dp_r, bat_r, hproj_r, hs_r, dinv_r, a_r, cnt_r):
    i = pl.program_id(0)
    h = jnp.dot(x_r[...], w_r[...], preferred_element_type=jnp.float32,
                precision=_HI)
    dp = dp_r[...]
    deg = 1.0 + dp[0, :, 0:1] + dp[1, :, 0:1]
    dinv = lax.rsqrt(deg)
    hproj_r[...] = h
    hs = h * dinv
    hs_r[...] = jnp.stack([hs[:, :64], hs[:, 64:]], axis=0)
    dinv_r[...] = jnp.broadcast_to(dinv, (BLK, 128))
    gids = lax.broadcasted_iota(jnp.int32, (G, BLK), 0)
    colid = lax.broadcasted_iota(jnp.int32, (G, BLK), 1) + i * BLK
    a = jnp.where((bat_r[...] == gids) & (colid < N), 1.0, 0.0)
    a_r[...] = a

    @pl.when(i == 0)
    def _():
        cnt_r[...] = jnp.zeros_like(cnt_r)

    cnt_r[...] += jnp.broadcast_to(jnp.sum(a, axis=1, keepdims=True), (G, 128))


def _tc_prep(xp, W0, degp, batp):
    return pl.pallas_call(
        _k0_body,
        grid=(GRID,),
        in_specs=[
            pl.BlockSpec((BLK, 128), lambda i: (i, 0)),
            pl.BlockSpec((128, 128), lambda i: (0, 0)),
            pl.BlockSpec((2, BLK, 16), lambda i: (0, i, 0)),
            pl.BlockSpec((1, BLK), lambda i: (0, i)),
        ],
        out_specs=[
            pl.BlockSpec((BLK, 128), lambda i: (i, 0)),
            pl.BlockSpec((2, BLK, 64), lambda i: (0, i, 0)),
            pl.BlockSpec((BLK, 128), lambda i: (i, 0)),
            pl.BlockSpec((G, BLK), lambda i: (0, i)),
            pl.BlockSpec((G, 128), lambda i: (0, 0)),
        ],
        out_shape=[
            jax.ShapeDtypeStruct((NPAD, 128), jnp.float32),
            jax.ShapeDtypeStruct((2, NPAD, 64), jnp.float32),
            jax.ShapeDtypeStruct((NPAD, 128), jnp.float32),
            jax.ShapeDtypeStruct((G, NPAD), jnp.float32),
            jax.ShapeDtypeStruct((G, 128), jnp.float32),
        ],
    )(xp, W0, degp, batp)


def _c1_body(part_r, hproj_r, dinv_r, b_r, a_r, m_r, s1_r, s2_r):
    i = pl.program_id(0)
    part = part_r[...]
    dinv = dinv_r[...]
    s = jnp.concatenate([part[0], part[1]], axis=1)
    m = dinv * s + dinv * dinv * hproj_r[...] + b_r[...]
    m_r[...] = m
    a = a_r[...]

    @pl.when(i == 0)
    def _():
        s1_r[...] = jnp.zeros_like(s1_r)
        s2_r[...] = jnp.zeros_like(s2_r)

    s1_r[...] += jnp.dot(a, m, preferred_element_type=jnp.float32, precision=_HI)
    s2_r[...] += jnp.dot(a, m * m, preferred_element_type=jnp.float32,
                         precision=_HI)


def _tc_c1(part, hproj, dinv, b2d, A):
    return pl.pallas_call(
        _c1_body,
        grid=(GRID,),
        in_specs=[
            pl.BlockSpec((2, BLK, 64), lambda i: (0, i, 0)),
            pl.BlockSpec((BLK, 128), lambda i: (i, 0)),
            pl.BlockSpec((BLK, 128), lambda i: (i, 0)),
            pl.BlockSpec((1, 128), lambda i: (0, 0)),
            pl.BlockSpec((G, BLK), lambda i: (0, i)),
        ],
        out_specs=[
            pl.BlockSpec((BLK, 128), lambda i: (i, 0)),
            pl.BlockSpec((G, 128), lambda i: (0, 0)),
            pl.BlockSpec((G, 128), lambda i: (0, 0)),
        ],
        out_shape=[
            jax.ShapeDtypeStruct((NPAD, 128), jnp.float32),
            jax.ShapeDtypeStruct((G, 128), jnp.float32),
            jax.ShapeDtypeStruct((G, 128), jnp.float32),
        ],
    )(part, hproj, dinv, b2d, A)


def _norm_block(m_r, s1_r, s2_r, cnt_r, a_r, gw_r, gb_r, gm_r):
    cnt = jnp.maximum(cnt_r[...], 1.0)
    mean = s1_r[...] / cnt
    msq = s2_r[...] / cnt
    gm = gm_r[...]
    var = msq - mean * mean * gm * (2.0 - gm)
    istd = lax.rsqrt(var + 1e-5)
    a = a_r[...]
    dn = (((0,), (0,)), ((), ()))
    mg = lax.dot_general(a, mean * gm, dn, precision=_HI,
                         preferred_element_type=jnp.float32)
    sb = lax.dot_general(a, istd, dn, precision=_HI,
                         preferred_element_type=jnp.float32)
    out = (m_r[...] - mg) * sb * gw_r[...] + gb_r[...]
    return jnp.maximum(out, 0.0), a


def _c2_body(m_r, s1_r, s2_r, cnt_r, a_r, gw_r, gb_r, gm_r, dinv_r, wn_r,
             hpn_r, hsn_r, gsum_r):
    i = pl.program_id(0)
    h, a = _norm_block(m_r, s1_r, s2_r, cnt_r, a_r, gw_r, gb_r, gm_r)

    @pl.when(i == 0)
    def _():
        gsum_r[...] = jnp.zeros_like(gsum_r)

    gsum_r[...] += jnp.dot(a, h, preferred_element_type=jnp.float32,
                           precision=_HI)
    hpn = jnp.dot(h, wn_r[...], preferred_element_type=jnp.float32,
                  precision=_HI)
    hpn_r[...] = hpn
    hsn = hpn * dinv_r[...]
    hsn_r[...] = jnp.stack([hsn[:, :64], hsn[:, 64:]], axis=0)


def _c2_last_body(m_r, s1_r, s2_r, cnt_r, a_r, gw_r, gb_r, gm_r,
                  h_r, gsum_r):
    i = pl.program_id(0)
    h, a = _norm_block(m_r, s1_r, s2_r, cnt_r, a_r, gw_r, gb_r, gm_r)
    h_r[...] = h

    @pl.when(i == 0)
    def _():
        gsum_r[...] = jnp.zeros_like(gsum_r)

    gsum_r[...] += jnp.dot(a, h, preferred_element_type=jnp.float32,
                           precision=_HI)


_C2_IN_SPECS = [
    pl.BlockSpec((BLK, 128), lambda i: (i, 0)),
    pl.BlockSpec((G, 128), lambda i: (0, 0)),
    pl.BlockSpec((G, 128), lambda i: (0, 0)),
    pl.BlockSpec((G, 128), lambda i: (0, 0)),
    pl.BlockSpec((G, BLK), lambda i: (0, i)),
    pl.BlockSpec((1, 128), lambda i: (0, 0)),
    pl.BlockSpec((1, 128), lambda i: (0, 0)),
    pl.BlockSpec((1, 128), lambda i: (0, 0)),
]


def _tc_c2(m, S1, S2, counts, A, gw, gb, gm, dinv, Wn):
    return pl.pallas_call(
        _c2_body,
        grid=(GRID,),
        in_specs=_C2_IN_SPECS + [
            pl.BlockSpec((BLK, 128), lambda i: (i, 0)),
            pl.BlockSpec((128, 128), lambda i: (0, 0)),
        ],
        out_specs=[
            pl.BlockSpec((BLK, 128), lambda i: (i, 0)),
            pl.BlockSpec((2, BLK, 64), lambda i: (0, i, 0)),
            pl.BlockSpec((G, 128), lambda i: (0, 0)),
        ],
        out_shape=[
            jax.ShapeDtypeStruct((NPAD, 128), jnp.float32),
            jax.ShapeDtypeStruct((2, NPAD, 64), jnp.float32),
            jax.ShapeDtypeStruct((G, 128), jnp.float32),
        ],
    )(m, S1, S2, counts, A, gw, gb, gm, dinv, Wn)


def _tc_c2_last(m, S1, S2, counts, A, gw, gb, gm):
    return pl.pallas_call(
        _c2_last_body,
        grid=(GRID,),
        in_specs=_C2_IN_SPECS,
        out_specs=[
            pl.BlockSpec((BLK, 128), lambda i: (i, 0)),
            pl.BlockSpec((G, 128), lambda i: (0, 0)),
        ],
        out_shape=[
            jax.ShapeDtypeStruct((NPAD, 128), jnp.float32),
            jax.ShapeDtypeStruct((G, 128), jnp.float32),
        ],
    )(m, S1, S2, counts, A, gw, gb, gm)


def _head_body(g1_r, g2_r, g3_r, cnt_r, w1_r, b1_r, w2_r, b2_r, out_r):
    cnt = jnp.maximum(cnt_r[...], 1.0)
    pooled = jnp.concatenate(
        [g1_r[...] / cnt, g2_r[...] / cnt, g3_r[...] / cnt], axis=1)
    z = jnp.dot(pooled, w1_r[...], preferred_element_type=jnp.float32,
                precision=_HI) + b1_r[...]
    z = jnp.maximum(z, 0.0)
    z = jnp.dot(z, w2_r[...], preferred_element_type=jnp.float32,
                precision=_HI) + b2_r[...]
    mask = lax.broadcasted_iota(jnp.int32, (G, 128), 1) < C
    z = jnp.where(mask, z, -jnp.inf)
    zmax = jnp.max(z, axis=1, keepdims=True)
    ez = jnp.where(mask, jnp.exp(z - zmax), 0.0)
    lse = jnp.log(jnp.sum(ez, axis=1, keepdims=True)) + zmax
    out_r[...] = jnp.where(mask, z - lse, 0.0)


def _tc_head(g1, g2, g3, counts, Wd1, bd1, Wd2p, bd2p):
    return pl.pallas_call(
        _head_body,
        grid=(1,),
        in_specs=[
            pl.BlockSpec((G, 128), lambda i: (0, 0)),
            pl.BlockSpec((G, 128), lambda i: (0, 0)),
            pl.BlockSpec((G, 128), lambda i: (0, 0)),
            pl.BlockSpec((G, 128), lambda i: (0, 0)),
            pl.BlockSpec((3 * H, 3 * H), lambda i: (0, 0)),
            pl.BlockSpec((1, 3 * H), lambda i: (0, 0)),
            pl.BlockSpec((3 * H, 128), lambda i: (0, 0)),
            pl.BlockSpec((1, 128), lambda i: (0, 0)),
        ],
        out_specs=pl.BlockSpec((G, 128), lambda i: (0, 0)),
        out_shape=jax.ShapeDtypeStruct((G, 128), jnp.float32),
    )(g1, g2, g3, counts, Wd1, bd1, Wd2p, bd2p)


def kernel(x, edge_index, batch, W0, b0, gnw0, gnb0, gnm0, W1, b1, gnw1,
           gnb1, gnm1, W2, b2, gnw2, gnb2, gnm2, Wd1, bd1, Wd2, bd2):
    f32 = jnp.float32
    xp = jnp.pad(x, ((0, NPAD - N), (0, 0)))
    srcw = jnp.pad(edge_index[0].astype(jnp.int32), (0, EPAD - E),
                   constant_values=NPAD - 1).reshape(NW, CH, K)
    dstw = jnp.pad(edge_index[1].astype(jnp.int32), (0, EPAD - E),
                   constant_values=NPAD - 1).reshape(NW, CH, K)
    batp = jnp.pad(batch.astype(jnp.int32), (0, NPAD - N)).reshape(1, NPAD)

    zeros64 = jnp.zeros((NPAD, 64), f32)
    zeros16 = jnp.zeros((NPAD, 16), f32)
    ones16 = jnp.ones((K, 16), f32)

    degp = _sc_deg(dstw, ones16, zeros16)
    hproj, hs, dinv, A, counts = _tc_prep(xp, W0, degp, batp)

    layers = [
        (b0, gnw0, gnb0, gnm0, W1),
        (b1, gnw1, gnb1, gnm1, W2),
        (b2, gnw2, gnb2, gnm2, None),
    ]
    gsums = []
    h_last = None
    for (b, gw, gb, gm, Wn) in layers:
        part = _sc_scatter(hs, srcw, dstw, zeros64)
        m, S1, S2 = _tc_c1(part, hproj, dinv, b.reshape(1, 128), A)
        args = (m, S1, S2, counts, A, gw.reshape(1, 128), gb.reshape(1, 128),
                gm.reshape(1, 128))
        if Wn is not None:
            hproj, hs, g = _tc_c2(*args, dinv, Wn)
        else:
            h_last, g = _tc_c2_last(*args)
        gsums.append(g)

    Wd2p = jnp.pad(Wd2, ((0, 0), (0, 128 - C)))
    bd2p = jnp.pad(bd2, (0, 128 - C)).reshape(1, 128)
    zp = _tc_head(gsums[0], gsums[1], gsums[2], counts,
                  Wd1, bd1.reshape(1, 3 * H), Wd2p, bd2p)
    return (h_last[:N], zp[:, :C])


# speedup vs baseline: 21.5797x; 2.0969x over previous
import functools

import jax
import jax.numpy as jnp
from jax import lax
from jax.experimental import pallas as pl
from jax.experimental.pallas import tpu as pltpu
from jax.experimental.pallas import tpu_sc as plsc

N = 10000
NPAD = 10240
E = 320000
D = 128
H = 128
G = 64
C = 32

NW = 16
K = 128
CH = 160
CHC = CH // 2
EW = CH * K
EPAD = NW * EW
NB = 4
RPT = NPAD // 16

BLK = 1024
GRID = NPAD // BLK

_HI = lax.Precision.HIGHEST


def _mesh():
    return plsc.VectorSubcoreMesh(core_axis_name="c", subcore_axis_name="s",
                                  num_cores=2, num_subcores=16)


_SC_PARAMS = pltpu.CompilerParams(use_tc_tiling_on_sc=False)


@functools.cache
def _build_sc_deg():
    return functools.partial(
        pl.kernel,
        out_type=jax.ShapeDtypeStruct((2, NPAD, 16), jnp.float32),
        mesh=_mesh(),
        scratch_types=[
            pltpu.VMEM((CH, K), jnp.int32),
            pltpu.VMEM((K, 16), jnp.float32),
            pltpu.VMEM_SHARED((NPAD, 16), jnp.float32),
        ],
        compiler_params=_SC_PARAMS,
    )(_sc_deg_body)


def _sc_deg(dstw, ones16, zeros16):
    return _build_sc_deg()(dstw, ones16, zeros16)


def _sc_deg_body(dst_hbm, ones_hbm, z_hbm, out_hbm, idx_v, ones_v, acc):
    core = lax.axis_index("c")
    sub = lax.axis_index("s")

    @pl.when(sub == 0)
    def _():
        pltpu.sync_copy(z_hbm, acc)

    plsc.subcore_barrier()
    pltpu.sync_copy(dst_hbm.at[sub], idx_v)
    pltpu.sync_copy(ones_hbm, ones_v)
    off = core * CHC

    def body(j, carry):
        pltpu.sync_copy(ones_v, acc.at[idx_v.at[off + j]], add=True)
        return carry

    lax.fori_loop(0, CHC, body, 0)
    plsc.subcore_barrier()
    start = sub * RPT
    pltpu.sync_copy(acc.at[pl.ds(start, RPT)], out_hbm.at[core].at[pl.ds(start, RPT)])


@functools.cache
def _build_sc_scatter():
    return functools.partial(
        pl.kernel,
        out_type=jax.ShapeDtypeStruct((2, NPAD, 64), jnp.float32),
        mesh=_mesh(),
        scratch_types=[
            pltpu.VMEM((CH, K), jnp.int32),
            pltpu.VMEM((CH, K), jnp.int32),
            [pltpu.VMEM((K, 64), jnp.float32) for _ in range(NB)],
            [pltpu.SemaphoreType.DMA for _ in range(NB)],
            [pltpu.SemaphoreType.DMA for _ in range(NB)],
            pltpu.VMEM_SHARED((NPAD, 64), jnp.float32),
        ],
        compiler_params=_SC_PARAMS,
    )(_sc_scatter_body)


def _sc_scatter(hs2, srcw, dstw, zeros64):
    return _build_sc_scatter()(hs2, srcw, dstw, zeros64)


def _sc_scatter_body(hs_hbm, src_hbm, dst_hbm, z_hbm, out_hbm,
                     sidx, didx, bufs, gsems, ssems, acc):
    core = lax.axis_index("c")
    sub = lax.axis_index("s")

    @pl.when(sub == 0)
    def _():
        pltpu.sync_copy(z_hbm, acc)

    plsc.subcore_barrier()
    pltpu.sync_copy(src_hbm.at[sub], sidx)
    pltpu.sync_copy(dst_hbm.at[sub], didx)
    table = hs_hbm.at[core]

    def _drain(buf, sem):
        pltpu.make_async_copy(z_hbm.at[pl.ds(0, K)], buf, sem).wait()

    for b in range(NB):
        pltpu.async_copy(table.at[sidx.at[b]], bufs[b], gsems[b])

    groups = CH // NB

    def body(j, carry):
        for b in range(NB):
            _drain(bufs[b], gsems[b])
            pltpu.async_copy(bufs[b], acc.at[didx.at[j * NB + b]], ssems[b],
                             add=True)
        for b in range(NB):
            _drain(bufs[b], ssems[b])

            @pl.when(j + 1 < groups)
            def _():
                pltpu.async_copy(table.at[sidx.at[(j + 1) * NB + b]],
                                 bufs[b], gsems[b])

        return carry

    lax.fori_loop(0, groups, body, 0)
    plsc.subcore_barrier()
    start = sub * RPT
    pltpu.sync_copy(acc.at[pl.ds(start, RPT)], out_hbm.at[core].at[pl.ds(start, RPT)])


def _k0_body(x_r, w_r, dp_r, bat_r, hproj_r, hs_r, dinv_r, a_r, cnt_r):
    i = pl.program_id(0)
    h = jnp.dot(x_r[...], w_r[...], preferred_element_type=jnp.float32,
                precision=_HI)
    dp = dp_r[...]
    deg = 1.0 + dp[0, :, 0:1] + dp[1, :, 0:1]
    dinv = lax.rsqrt(deg)
    hproj_r[...] = h
    hs = h * dinv
    hs_r[...] = jnp.stack([hs[:, :64], hs[:, 64:]], axis=0)
    dinv_r[...] = jnp.broadcast_to(dinv, (BLK, 128))
    gids = lax.broadcasted_iota(jnp.int32, (G, BLK), 0)
    colid = lax.broadcasted_iota(jnp.int32, (G, BLK), 1) + i * BLK
    a = jnp.where((bat_r[...] == gids) & (colid < N), 1.0, 0.0)
    a_r[...] = a

    @pl.when(i == 0)
    def _():
        cnt_r[...] = jnp.zeros_like(cnt_r)

    cnt_r[...] += jnp.broadcast_to(jnp.sum(a, axis=1, keepdims=True), (G, 128))


def _tc_prep(xp, W0, degp, batp):
    return pl.pallas_call(
        _k0_body,
        grid=(GRID,),
        in_specs=[
            pl.BlockSpec((BLK, 128), lambda i: (i, 0)),
            pl.BlockSpec((128, 128), lambda i: (0, 0)),
            pl.BlockSpec((2, BLK, 16), lambda i: (0, i, 0)),
            pl.BlockSpec((1, BLK), lambda i: (0, i)),
        ],
        out_specs=[
            pl.BlockSpec((BLK, 128), lambda i: (i, 0)),
            pl.BlockSpec((2, BLK, 64), lambda i: (0, i, 0)),
            pl.BlockSpec((BLK, 128), lambda i: (i, 0)),
            pl.BlockSpec((G, BLK), lambda i: (0, i)),
            pl.BlockSpec((G, 128), lambda i: (0, 0)),
        ],
        out_shape=[
            jax.ShapeDtypeStruct((NPAD, 128), jnp.float32),
            jax.ShapeDtypeStruct((2, NPAD, 64), jnp.float32),
            jax.ShapeDtypeStruct((NPAD, 128), jnp.float32),
            jax.ShapeDtypeStruct((G, NPAD), jnp.float32),
            jax.ShapeDtypeStruct((G, 128), jnp.float32),
        ],
    )(xp, W0, degp, batp)


def _c1_body(part_r, hproj_r, dinv_r, b_r, a_r, m_r, s1_r, s2_r):
    i = pl.program_id(0)
    part = part_r[...]
    dinv = dinv_r[...]
    s = jnp.concatenate([part[0], part[1]], axis=1)
    m = dinv * s + dinv * dinv * hproj_r[...] + b_r[...]
    m_r[...] = m
    a = a_r[...]

    @pl.when(i == 0)
    def _():
        s1_r[...] = jnp.zeros_like(s1_r)
        s2_r[...] = jnp.zeros_like(s2_r)

    s1_r[...] += jnp.dot(a, m, preferred_element_type=jnp.float32, precision=_HI)
    s2_r[...] += jnp.dot(a, m * m, preferred_element_type=jnp.float32,
                         precision=_HI)


def _tc_c1(part, hproj, dinv, b2d, A):
    return pl.pallas_call(
        _c1_body,
        grid=(GRID,),
        in_specs=[
            pl.BlockSpec((2, BLK, 64), lambda i: (0, i, 0)),
            pl.BlockSpec((BLK, 128), lambda i: (i, 0)),
            pl.BlockSpec((BLK, 128), lambda i: (i, 0)),
            pl.BlockSpec((1, 128), lambda i: (0, 0)),
            pl.BlockSpec((G, BLK), lambda i: (0, i)),
        ],
        out_specs=[
            pl.BlockSpec((BLK, 128), lambda i: (i, 0)),
            pl.BlockSpec((G, 128), lambda i: (0, 0)),
            pl.BlockSpec((G, 128), lambda i: (0, 0)),
        ],
        out_shape=[
            jax.ShapeDtypeStruct((NPAD, 128), jnp.float32),
            jax.ShapeDtypeStruct((G, 128), jnp.float32),
            jax.ShapeDtypeStruct((G, 128), jnp.float32),
        ],
    )(part, hproj, dinv, b2d, A)


def _norm_block(m_r, s1_r, s2_r, cnt_r, a_r, gw_r, gb_r, gm_r):
    cnt = jnp.maximum(cnt_r[...], 1.0)
    mean = s1_r[...] / cnt
    msq = s2_r[...] / cnt
    gm = gm_r[...]
    var = msq - mean * mean * gm * (2.0 - gm)
    istd = lax.rsqrt(var + 1e-5)
    a = a_r[...]
    dn = (((0,), (0,)), ((), ()))
    mg = lax.dot_general(a, mean * gm, dn, precision=_HI,
                         preferred_element_type=jnp.float32)
    sb = lax.dot_general(a, istd, dn, precision=_HI,
                         preferred_element_type=jnp.float32)
    out = (m_r[...] - mg) * sb * gw_r[...] + gb_r[...]
    return jnp.maximum(out, 0.0), a


def _c2_body(m_r, s1_r, s2_r, cnt_r, a_r, gw_r, gb_r, gm_r, dinv_r, wn_r,
             hpn_r, hsn_r, gsum_r):
    i = pl.program_id(0)
    h, a = _norm_block(m_r, s1_r, s2_r, cnt_r, a_r, gw_r, gb_r, gm_r)

    @pl.when(i == 0)
    def _():
        gsum_r[...] = jnp.zeros_like(gsum_r)

    gsum_r[...] += jnp.dot(a, h, preferred_element_type=jnp.float32,
                           precision=_HI)
    hpn = jnp.dot(h, wn_r[...], preferred_element_type=jnp.float32,
                  precision=_HI)
    hpn_r[...] = hpn
    hsn = hpn * dinv_r[...]
    hsn_r[...] = jnp.stack([hsn[:, :64], hsn[:, 64:]], axis=0)


def _c2_last_body(m_r, s1_r, s2_r, cnt_r, a_r, gw_r, gb_r, gm_r,
                  h_r, gsum_r):
    i = pl.program_id(0)
    h, a = _norm_block(m_r, s1_r, s2_r, cnt_r, a_r, gw_r, gb_r, gm_r)
    h_r[...] = h

    @pl.when(i == 0)
    def _():
        gsum_r[...] = jnp.zeros_like(gsum_r)

    gsum_r[...] += jnp.dot(a, h, preferred_element_type=jnp.float32,
                           precision=_HI)


_C2_IN_SPECS = [
    pl.BlockSpec((BLK, 128), lambda i: (i, 0)),
    pl.BlockSpec((G, 128), lambda i: (0, 0)),
    pl.BlockSpec((G, 128), lambda i: (0, 0)),
    pl.BlockSpec((G, 128), lambda i: (0, 0)),
    pl.BlockSpec((G, BLK), lambda i: (0, i)),
    pl.BlockSpec((1, 128), lambda i: (0, 0)),
    pl.BlockSpec((1, 128), lambda i: (0, 0)),
    pl.BlockSpec((1, 128), lambda i: (0, 0)),
]


def _tc_c2(m, S1, S2, counts, A, gw, gb, gm, dinv, Wn):
    return pl.pallas_call(
        _c2_body,
        grid=(GRID,),
        in_specs=_C2_IN_SPECS + [
            pl.BlockSpec((BLK, 128), lambda i: (i, 0)),
            pl.BlockSpec((128, 128), lambda i: (0, 0)),
        ],
        out_specs=[
            pl.BlockSpec((BLK, 128), lambda i: (i, 0)),
            pl.BlockSpec((2, BLK, 64), lambda i: (0, i, 0)),
            pl.BlockSpec((G, 128), lambda i: (0, 0)),
        ],
        out_shape=[
            jax.ShapeDtypeStruct((NPAD, 128), jnp.float32),
            jax.ShapeDtypeStruct((2, NPAD, 64), jnp.float32),
            jax.ShapeDtypeStruct((G, 128), jnp.float32),
        ],
    )(m, S1, S2, counts, A, gw, gb, gm, dinv, Wn)


def _tc_c2_last(m, S1, S2, counts, A, gw, gb, gm):
    return pl.pallas_call(
        _c2_last_body,
        grid=(GRID,),
        in_specs=_C2_IN_SPECS,
        out_specs=[
            pl.BlockSpec((BLK, 128), lambda i: (i, 0)),
            pl.BlockSpec((G, 128), lambda i: (0, 0)),
        ],
        out_shape=[
            jax.ShapeDtypeStruct((NPAD, 128), jnp.float32),
            jax.ShapeDtypeStruct((G, 128), jnp.float32),
        ],
    )(m, S1, S2, counts, A, gw, gb, gm)


def _head_body(g1_r, g2_r, g3_r, cnt_r, w1_r, b1_r, w2_r, b2_r, out_r):
    cnt = jnp.maximum(cnt_r[...], 1.0)
    pooled = jnp.concatenate(
        [g1_r[...] / cnt, g2_r[...] / cnt, g3_r[...] / cnt], axis=1)
    z = jnp.dot(pooled, w1_r[...], preferred_element_type=jnp.float32,
                precision=_HI) + b1_r[...]
    z = jnp.maximum(z, 0.0)
    z = jnp.dot(z, w2_r[...], preferred_element_type=jnp.float32,
                precision=_HI) + b2_r[...]
    mask = lax.broadcasted_iota(jnp.int32, (G, 128), 1) < C
    z = jnp.where(mask, z, -jnp.inf)
    zmax = jnp.max(z, axis=1, keepdims=True)
    ez = jnp.where(mask, jnp.exp(z - zmax), 0.0)
    lse = jnp.log(jnp.sum(ez, axis=1, keepdims=True)) + zmax
    out_r[...] = jnp.where(mask, z - lse, 0.0)


def _tc_head(g1, g2, g3, counts, Wd1, bd1, Wd2p, bd2p):
    return pl.pallas_call(
        _head_body,
        grid=(1,),
        in_specs=[
            pl.BlockSpec((G, 128), lambda i: (0, 0)),
            pl.BlockSpec((G, 128), lambda i: (0, 0)),
            pl.BlockSpec((G, 128), lambda i: (0, 0)),
            pl.BlockSpec((G, 128), lambda i: (0, 0)),
            pl.BlockSpec((3 * H, 3 * H), lambda i: (0, 0)),
            pl.BlockSpec((1, 3 * H), lambda i: (0, 0)),
            pl.BlockSpec((3 * H, 128), lambda i: (0, 0)),
            pl.BlockSpec((1, 128), lambda i: (0, 0)),
        ],
        out_specs=pl.BlockSpec((G, 128), lambda i: (0, 0)),
        out_shape=jax.ShapeDtypeStruct((G, 128), jnp.float32),
    )(g1, g2, g3, counts, Wd1, bd1, Wd2p, bd2p)


def kernel(x, edge_index, batch, W0, b0, gnw0, gnb0, gnm0, W1, b1, gnw1,
           gnb1, gnm1, W2, b2, gnw2, gnb2, gnm2, Wd1, bd1, Wd2, bd2):
    f32 = jnp.float32
    xp = jnp.pad(x, ((0, NPAD - N), (0, 0)))
    pad_ids = N + jnp.arange(EPAD - E, dtype=jnp.int32) % (NPAD - N)
    srcw = jnp.concatenate(
        [edge_index[0].astype(jnp.int32), pad_ids]).reshape(NW, CH, K)
    dstw = jnp.concatenate(
        [edge_index[1].astype(jnp.int32), pad_ids]).reshape(NW, CH, K)
    batp = jnp.pad(batch.astype(jnp.int32), (0, NPAD - N)).reshape(1, NPAD)

    zeros64 = jnp.zeros((NPAD, 64), f32)
    zeros16 = jnp.zeros((NPAD, 16), f32)
    ones16 = jnp.ones((K, 16), f32)

    degp = _sc_deg(dstw, ones16, zeros16)
    hproj, hs, dinv, A, counts = _tc_prep(xp, W0, degp, batp)

    layers = [
        (b0, gnw0, gnb0, gnm0, W1),
        (b1, gnw1, gnb1, gnm1, W2),
        (b2, gnw2, gnb2, gnm2, None),
    ]
    gsums = []
    h_last = None
    for (b, gw, gb, gm, Wn) in layers:
        part = _sc_scatter(hs, srcw, dstw, zeros64)
        m, S1, S2 = _tc_c1(part, hproj, dinv, b.reshape(1, 128), A)
        args = (m, S1, S2, counts, A, gw.reshape(1, 128), gb.reshape(1, 128),
                gm.reshape(1, 128))
        if Wn is not None:
            hproj, hs, g = _tc_c2(*args, dinv, Wn)
        else:
            h_last, g = _tc_c2_last(*args)
        gsums.append(g)

    Wd2p = jnp.pad(Wd2, ((0, 0), (0, 128 - C)))
    bd2p = jnp.pad(bd2, (0, 128 - C)).reshape(1, 128)
    zp = _tc_head(gsums[0], gsums[1], gsums[2], counts,
                  Wd1, bd1.reshape(1, 3 * H), Wd2p, bd2p)
    return (h_last[:N], zp[:, :C])


# speedup vs baseline: 21.7027x; 1.0057x over previous
import functools

import jax
import jax.numpy as jnp
from jax import lax
from jax.experimental import pallas as pl
from jax.experimental.pallas import tpu as pltpu
from jax.experimental.pallas import tpu_sc as plsc

N = 10000
NPAD = 10240
E = 320000
D = 128
H = 128
G = 64
C = 32

NW = 16
K = 128
CH = 160
CHC = CH // 2
EW = CH * K
EPAD = NW * EW
NB = 5
RPT = NPAD // 16

BLK = 1024
GRID = NPAD // BLK

_HI = lax.Precision.HIGHEST


def _mesh():
    return plsc.VectorSubcoreMesh(core_axis_name="c", subcore_axis_name="s",
                                  num_cores=2, num_subcores=16)


_SC_PARAMS = pltpu.CompilerParams(use_tc_tiling_on_sc=False)


@functools.cache
def _build_sc_deg():
    return functools.partial(
        pl.kernel,
        out_type=jax.ShapeDtypeStruct((2, NPAD, 16), jnp.float32),
        mesh=_mesh(),
        scratch_types=[
            pltpu.VMEM((CH, K), jnp.int32),
            pltpu.VMEM((K, 16), jnp.float32),
            pltpu.VMEM_SHARED((NPAD, 16), jnp.float32),
        ],
        compiler_params=_SC_PARAMS,
    )(_sc_deg_body)


def _sc_deg(dstw, ones16, zeros16):
    return _build_sc_deg()(dstw, ones16, zeros16)


def _sc_deg_body(dst_hbm, ones_hbm, z_hbm, out_hbm, idx_v, ones_v, acc):
    core = lax.axis_index("c")
    sub = lax.axis_index("s")

    @pl.when(sub == 0)
    def _():
        pltpu.sync_copy(z_hbm, acc)

    plsc.subcore_barrier()
    pltpu.sync_copy(dst_hbm.at[sub], idx_v)
    pltpu.sync_copy(ones_hbm, ones_v)
    off = core * CHC

    def body(j, carry):
        pltpu.sync_copy(ones_v, acc.at[idx_v.at[off + j]], add=True)
        return carry

    lax.fori_loop(0, CHC, body, 0)
    plsc.subcore_barrier()
    start = sub * RPT
    pltpu.sync_copy(acc.at[pl.ds(start, RPT)], out_hbm.at[core].at[pl.ds(start, RPT)])


@functools.cache
def _build_sc_scatter():
    return functools.partial(
        pl.kernel,
        out_type=jax.ShapeDtypeStruct((2, NPAD, 64), jnp.float32),
        mesh=_mesh(),
        scratch_types=[
            pltpu.VMEM((CH, K), jnp.int32),
            pltpu.VMEM((CH, K), jnp.int32),
            [pltpu.VMEM((K, 64), jnp.float32) for _ in range(NB)],
            [pltpu.SemaphoreType.DMA for _ in range(NB)],
            [pltpu.SemaphoreType.DMA for _ in range(NB)],
            pltpu.VMEM_SHARED((NPAD, 64), jnp.float32),
        ],
        compiler_params=_SC_PARAMS,
    )(_sc_scatter_body)


def _sc_scatter(hs2, srcw, dstw, zeros64):
    return _build_sc_scatter()(hs2, srcw, dstw, zeros64)


def _sc_scatter_body(hs_hbm, src_hbm, dst_hbm, z_hbm, out_hbm,
                     sidx, didx, bufs, gsems, ssems, acc):
    core = lax.axis_index("c")
    sub = lax.axis_index("s")

    @pl.when(sub == 0)
    def _():
        pltpu.sync_copy(z_hbm, acc)

    plsc.subcore_barrier()
    pltpu.sync_copy(src_hbm.at[sub], sidx)
    pltpu.sync_copy(dst_hbm.at[sub], didx)
    table = hs_hbm.at[core]

    def _drain(buf, sem):
        pltpu.make_async_copy(z_hbm.at[pl.ds(0, K)], buf, sem).wait()

    for b in range(NB):
        pltpu.async_copy(table.at[sidx.at[b]], bufs[b], gsems[b])

    groups = CH // NB

    def body(j, carry):
        for b in range(NB):
            _drain(bufs[b], gsems[b])
            pltpu.async_copy(bufs[b], acc.at[didx.at[j * NB + b]], ssems[b],
                             add=True)
        for b in range(NB):
            _drain(bufs[b], ssems[b])

            @pl.when(j + 1 < groups)
            def _():
                pltpu.async_copy(table.at[sidx.at[(j + 1) * NB + b]],
                                 bufs[b], gsems[b])

        return carry

    lax.fori_loop(0, groups, body, 0)
    plsc.subcore_barrier()
    start = sub * RPT
    pltpu.sync_copy(acc.at[pl.ds(start, RPT)], out_hbm.at[core].at[pl.ds(start, RPT)])


def _dinv_block(dp_r):
    dp = dp_r[...]
    return lax.rsqrt(1.0 + dp[0, :, 0:1] + dp[1, :, 0:1])


def _a_block(bat_r, i):
    gids = lax.broadcasted_iota(jnp.int32, (G, BLK), 0)
    colid = lax.broadcasted_iota(jnp.int32, (G, BLK), 1) + i * BLK
    return jnp.where((bat_r[...] == gids) & (colid < N), 1.0, 0.0)


def _k0_body(x_r, w_r, dp_r, bat_r, hproj_r, hs_r, cnt_r):
    i = pl.program_id(0)
    h = jnp.dot(x_r[...], w_r[...], preferred_element_type=jnp.float32,
                precision=_HI)
    dinv = _dinv_block(dp_r)
    hproj_r[...] = h
    hs = h * dinv
    hs_r[...] = jnp.stack([hs[:, :64], hs[:, 64:]], axis=0)
    a = _a_block(bat_r, i)

    @pl.when(i == 0)
    def _():
        cnt_r[...] = jnp.zeros_like(cnt_r)

    cnt_r[...] += jnp.broadcast_to(jnp.sum(a, axis=1, keepdims=True), (G, 128))


def _tc_prep(xp, W0, degp, batp):
    return pl.pallas_call(
        _k0_body,
        grid=(GRID,),
        in_specs=[
            pl.BlockSpec((BLK, 128), lambda i: (i, 0)),
            pl.BlockSpec((128, 128), lambda i: (0, 0)),
            pl.BlockSpec((2, BLK, 16), lambda i: (0, i, 0)),
            pl.BlockSpec((1, BLK), lambda i: (0, i)),
        ],
        out_specs=[
            pl.BlockSpec((BLK, 128), lambda i: (i, 0)),
            pl.BlockSpec((2, BLK, 64), lambda i: (0, i, 0)),
            pl.BlockSpec((G, 128), lambda i: (0, 0)),
        ],
        out_shape=[
            jax.ShapeDtypeStruct((NPAD, 128), jnp.float32),
            jax.ShapeDtypeStruct((2, NPAD, 64), jnp.float32),
            jax.ShapeDtypeStruct((G, 128), jnp.float32),
        ],
    )(xp, W0, degp, batp)


def _c1_body(part_r, hproj_r, dp_r, b_r, bat_r, m_r, s1_r, s2_r):
    i = pl.program_id(0)
    part = part_r[...]
    dinv = _dinv_block(dp_r)
    s = jnp.concatenate([part[0], part[1]], axis=1)
    m = dinv * s + dinv * dinv * hproj_r[...] + b_r[...]
    m_r[...] = m
    a = _a_block(bat_r, i)

    @pl.when(i == 0)
    def _():
        s1_r[...] = jnp.zeros_like(s1_r)
        s2_r[...] = jnp.zeros_like(s2_r)

    s1_r[...] += jnp.dot(a, m, preferred_element_type=jnp.float32, precision=_HI)
    s2_r[...] += jnp.dot(a, m * m, preferred_element_type=jnp.float32,
                         precision=_HI)


def _tc_c1(part, hproj, degp, b2d, batp):
    return pl.pallas_call(
        _c1_body,
        grid=(GRID,),
        in_specs=[
            pl.BlockSpec((2, BLK, 64), lambda i: (0, i, 0)),
            pl.BlockSpec((BLK, 128), lambda i: (i, 0)),
            pl.BlockSpec((2, BLK, 16), lambda i: (0, i, 0)),
            pl.BlockSpec((1, 128), lambda i: (0, 0)),
            pl.BlockSpec((1, BLK), lambda i: (0, i)),
        ],
        out_specs=[
            pl.BlockSpec((BLK, 128), lambda i: (i, 0)),
            pl.BlockSpec((G, 128), lambda i: (0, 0)),
            pl.BlockSpec((G, 128), lambda i: (0, 0)),
        ],
        out_shape=[
            jax.ShapeDtypeStruct((NPAD, 128), jnp.float32),
            jax.ShapeDtypeStruct((G, 128), jnp.float32),
            jax.ShapeDtypeStruct((G, 128), jnp.float32),
        ],
    )(part, hproj, degp, b2d, batp)


def _norm_block(m_r, s1_r, s2_r, cnt_r, a, gw_r, gb_r, gm_r):
    cnt = jnp.maximum(cnt_r[...], 1.0)
    mean = s1_r[...] / cnt
    msq = s2_r[...] / cnt
    gm = gm_r[...]
    var = msq - mean * mean * gm * (2.0 - gm)
    istd = lax.rsqrt(var + 1e-5)
    dn = (((0,), (0,)), ((), ()))
    mg = lax.dot_general(a, mean * gm, dn, precision=_HI,
                         preferred_element_type=jnp.float32)
    sb = lax.dot_general(a, istd, dn, precision=_HI,
                         preferred_element_type=jnp.float32)
    out = (m_r[...] - mg) * sb * gw_r[...] + gb_r[...]
    return jnp.maximum(out, 0.0)


def _c2_body(m_r, s1_r, s2_r, cnt_r, bat_r, gw_r, gb_r, gm_r, dp_r, wn_r,
             hpn_r, hsn_r, gsum_r):
    i = pl.program_id(0)
    a = _a_block(bat_r, i)
    h = _norm_block(m_r, s1_r, s2_r, cnt_r, a, gw_r, gb_r, gm_r)

    @pl.when(i == 0)
    def _():
        gsum_r[...] = jnp.zeros_like(gsum_r)

    gsum_r[...] += jnp.dot(a, h, preferred_element_type=jnp.float32,
                           precision=_HI)
    hpn = jnp.dot(h, wn_r[...], preferred_element_type=jnp.float32,
                  precision=_HI)
    hpn_r[...] = hpn
    hsn = hpn * _dinv_block(dp_r)
    hsn_r[...] = jnp.stack([hsn[:, :64], hsn[:, 64:]], axis=0)


def _c2_last_body(m_r, s1_r, s2_r, cnt_r, bat_r, gw_r, gb_r, gm_r,
                  h_r, gsum_r):
    i = pl.program_id(0)
    a = _a_block(bat_r, i)
    h = _norm_block(m_r, s1_r, s2_r, cnt_r, a, gw_r, gb_r, gm_r)
    h_r[...] = h

    @pl.when(i == 0)
    def _():
        gsum_r[...] = jnp.zeros_like(gsum_r)

    gsum_r[...] += jnp.dot(a, h, preferred_element_type=jnp.float32,
                           precision=_HI)


_C2_IN_SPECS = [
    pl.BlockSpec((BLK, 128), lambda i: (i, 0)),
    pl.BlockSpec((G, 128), lambda i: (0, 0)),
    pl.BlockSpec((G, 128), lambda i: (0, 0)),
    pl.BlockSpec((G, 128), lambda i: (0, 0)),
    pl.BlockSpec((1, BLK), lambda i: (0, i)),
    pl.BlockSpec((1, 128), lambda i: (0, 0)),
    pl.BlockSpec((1, 128), lambda i: (0, 0)),
    pl.BlockSpec((1, 128), lambda i: (0, 0)),
]


def _tc_c2(m, S1, S2, counts, batp, gw, gb, gm, degp, Wn):
    return pl.pallas_call(
        _c2_body,
        grid=(GRID,),
        in_specs=_C2_IN_SPECS + [
            pl.BlockSpec((2, BLK, 16), lambda i: (0, i, 0)),
            pl.BlockSpec((128, 128), lambda i: (0, 0)),
        ],
        out_specs=[
            pl.BlockSpec((BLK, 128), lambda i: (i, 0)),
            pl.BlockSpec((2, BLK, 64), lambda i: (0, i, 0)),
            pl.BlockSpec((G, 128), lambda i: (0, 0)),
        ],
        out_shape=[
            jax.ShapeDtypeStruct((NPAD, 128), jnp.float32),
            jax.ShapeDtypeStruct((2, NPAD, 64), jnp.float32),
            jax.ShapeDtypeStruct((G, 128), jnp.float32),
        ],
    )(m, S1, S2, counts, batp, gw, gb, gm, degp, Wn)


def _tc_c2_last(m, S1, S2, counts, batp, gw, gb, gm):
    return pl.pallas_call(
        _c2_last_body,
        grid=(GRID,),
        in_specs=_C2_IN_SPECS,
        out_specs=[
            pl.BlockSpec((BLK, 128), lambda i: (i, 0)),
            pl.BlockSpec((G, 128), lambda i: (0, 0)),
        ],
        out_shape=[
            jax.ShapeDtypeStruct((NPAD, 128), jnp.float32),
            jax.ShapeDtypeStruct((G, 128), jnp.float32),
        ],
    )(m, S1, S2, counts, batp, gw, gb, gm)


def _head_body(g1_r, g2_r, g3_r, cnt_r, w1_r, b1_r, w2_r, b2_r, out_r):
    cnt = jnp.maximum(cnt_r[...], 1.0)
    pooled = jnp.concatenate(
        [g1_r[...] / cnt, g2_r[...] / cnt, g3_r[...] / cnt], axis=1)
    z = jnp.dot(pooled, w1_r[...], preferred_element_type=jnp.float32,
                precision=_HI) + b1_r[...]
    z = jnp.maximum(z, 0.0)
    z = jnp.dot(z, w2_r[...], preferred_element_type=jnp.float32,
                precision=_HI) + b2_r[...]
    mask = lax.broadcasted_iota(jnp.int32, (G, 128), 1) < C
    z = jnp.where(mask, z, -jnp.inf)
    zmax = jnp.max(z, axis=1, keepdims=True)
    ez = jnp.where(mask, jnp.exp(z - zmax), 0.0)
    lse = jnp.log(jnp.sum(ez, axis=1, keepdims=True)) + zmax
    out_r[...] = jnp.where(mask, z - lse, 0.0)


def _tc_head(g1, g2, g3, counts, Wd1, bd1, Wd2p, bd2p):
    return pl.pallas_call(
        _head_body,
        grid=(1,),
        in_specs=[
            pl.BlockSpec((G, 128), lambda i: (0, 0)),
            pl.BlockSpec((G, 128), lambda i: (0, 0)),
            pl.BlockSpec((G, 128), lambda i: (0, 0)),
            pl.BlockSpec((G, 128), lambda i: (0, 0)),
            pl.BlockSpec((3 * H, 3 * H), lambda i: (0, 0)),
            pl.BlockSpec((1, 3 * H), lambda i: (0, 0)),
            pl.BlockSpec((3 * H, 128), lambda i: (0, 0)),
            pl.BlockSpec((1, 128), lambda i: (0, 0)),
        ],
        out_specs=pl.BlockSpec((G, 128), lambda i: (0, 0)),
        out_shape=jax.ShapeDtypeStruct((G, 128), jnp.float32),
    )(g1, g2, g3, counts, Wd1, bd1, Wd2p, bd2p)


def kernel(x, edge_index, batch, W0, b0, gnw0, gnb0, gnm0, W1, b1, gnw1,
           gnb1, gnm1, W2, b2, gnw2, gnb2, gnm2, Wd1, bd1, Wd2, bd2):
    f32 = jnp.float32
    xp = jnp.pad(x, ((0, NPAD - N), (0, 0)))
    pad_ids = N + jnp.arange(EPAD - E, dtype=jnp.int32) % (NPAD - N)
    srcw = jnp.concatenate(
        [edge_index[0].astype(jnp.int32), pad_ids]).reshape(NW, CH, K)
    dstw = jnp.concatenate(
        [edge_index[1].astype(jnp.int32), pad_ids]).reshape(NW, CH, K)
    batp = jnp.pad(batch.astype(jnp.int32), (0, NPAD - N)).reshape(1, NPAD)

    zeros64 = jnp.zeros((NPAD, 64), f32)
    zeros16 = jnp.zeros((NPAD, 16), f32)
    ones16 = jnp.ones((K, 16), f32)

    degp = _sc_deg(dstw, ones16, zeros16)
    hproj, hs, counts = _tc_prep(xp, W0, degp, batp)

    layers = [
        (b0, gnw0, gnb0, gnm0, W1),
        (b1, gnw1, gnb1, gnm1, W2),
        (b2, gnw2, gnb2, gnm2, None),
    ]
    gsums = []
    h_last = None
    for (b, gw, gb, gm, Wn) in layers:
        part = _sc_scatter(hs, srcw, dstw, zeros64)
        m, S1, S2 = _tc_c1(part, hproj, degp, b.reshape(1, 128), batp)
        args = (m, S1, S2, counts, batp, gw.reshape(1, 128),
                gb.reshape(1, 128), gm.reshape(1, 128))
        if Wn is not None:
            hproj, hs, g = _tc_c2(*args, degp, Wn)
        else:
            h_last, g = _tc_c2_last(*args)
        gsums.append(g)

    Wd2p = jnp.pad(Wd2, ((0, 0), (0, 128 - C)))
    bd2p = jnp.pad(bd2, (0, 128 - C)).reshape(1, 128)
    zp = _tc_head(gsums[0], gsums[1], gsums[2], counts,
                  Wd1, bd1.reshape(1, 3 * H), Wd2p, bd2p)
    return (h_last[:N], zp[:, :C])


# speedup vs baseline: 21.9206x; 1.0100x over previous
import functools

import jax
import jax.numpy as jnp
from jax import lax
from jax.experimental import pallas as pl
from jax.experimental.pallas import tpu as pltpu
from jax.experimental.pallas import tpu_sc as plsc

N = 10000
NPAD = 10240
E = 320000
D = 128
H = 128
G = 64
C = 32

NW = 16
K = 128
CH = 160
CHC = CH // 2
EW = CH * K
EPAD = NW * EW
NB = 5
RPT = NPAD // 16

BLK = 1024
GRID = NPAD // BLK

_HI = lax.Precision.HIGHEST


def _mesh():
    return plsc.VectorSubcoreMesh(core_axis_name="c", subcore_axis_name="s",
                                  num_cores=2, num_subcores=16)


_SC_PARAMS = pltpu.CompilerParams(use_tc_tiling_on_sc=False)


@functools.cache
def _build_sc_deg():
    return functools.partial(
        pl.kernel,
        out_type=jax.ShapeDtypeStruct((2, NPAD, 16), jnp.float32),
        mesh=_mesh(),
        scratch_types=[
            pltpu.VMEM((CH, K), jnp.int32),
            pltpu.VMEM((K, 16), jnp.float32),
            pltpu.VMEM_SHARED((NPAD, 16), jnp.float32),
        ],
        compiler_params=_SC_PARAMS,
    )(_sc_deg_body)


def _sc_deg(dstw, ones16, zeros16):
    return _build_sc_deg()(dstw, ones16, zeros16)


def _sc_deg_body(dst_hbm, ones_hbm, z_hbm, out_hbm, idx_v, ones_v, acc):
    core = lax.axis_index("c")
    sub = lax.axis_index("s")

    @pl.when(sub == 0)
    def _():
        pltpu.sync_copy(z_hbm, acc)

    plsc.subcore_barrier()
    pltpu.sync_copy(dst_hbm.at[sub], idx_v)
    pltpu.sync_copy(ones_hbm, ones_v)
    off = core * CHC

    def body(j, carry):
        pltpu.sync_copy(ones_v, acc.at[idx_v.at[off + j]], add=True)
        return carry

    lax.fori_loop(0, CHC, body, 0)
    plsc.subcore_barrier()
    start = sub * RPT
    pltpu.sync_copy(acc.at[pl.ds(start, RPT)], out_hbm.at[core].at[pl.ds(start, RPT)])


@functools.cache
def _build_sc_scatter():
    return functools.partial(
        pl.kernel,
        out_type=jax.ShapeDtypeStruct((2, NPAD, 64), jnp.float32),
        mesh=_mesh(),
        scratch_types=[
            pltpu.VMEM((CH, K), jnp.int32),
            pltpu.VMEM((CH, K), jnp.int32),
            [pltpu.VMEM((K, 64), jnp.float32) for _ in range(NB)],
            [pltpu.SemaphoreType.DMA for _ in range(NB)],
            [pltpu.SemaphoreType.DMA for _ in range(NB)],
            pltpu.VMEM_SHARED((NPAD, 64), jnp.float32),
        ],
        compiler_params=_SC_PARAMS,
    )(_sc_scatter_body)


def _sc_scatter(hs2, srcw, dstw, zeros64):
    return _build_sc_scatter()(hs2, srcw, dstw, zeros64)


def _sc_scatter_body(hs_hbm, src_hbm, dst_hbm, z_hbm, out_hbm,
                     sidx, didx, bufs, gsems, ssems, acc):
    core = lax.axis_index("c")
    sub = lax.axis_index("s")

    @pl.when(sub == 0)
    def _():
        pltpu.sync_copy(z_hbm, acc)

    plsc.subcore_barrier()
    pltpu.sync_copy(src_hbm.at[sub], sidx)
    pltpu.sync_copy(dst_hbm.at[sub], didx)
    table = hs_hbm.at[core]

    def _drain(buf, sem):
        pltpu.make_async_copy(z_hbm.at[pl.ds(0, K)], buf, sem).wait()

    for b in range(NB):
        pltpu.async_copy(table.at[sidx.at[b]], bufs[b], gsems[b])

    groups = CH // NB

    def body(j, carry):
        for b in range(NB):
            _drain(bufs[b], gsems[b])
            pltpu.async_copy(bufs[b], acc.at[didx.at[j * NB + b]], ssems[b],
                             add=True)
        for b in range(NB):
            _drain(bufs[b], ssems[b])

            @pl.when(j + 1 < groups)
            def _():
                pltpu.async_copy(table.at[sidx.at[(j + 1) * NB + b]],
                                 bufs[b], gsems[b])

        return carry

    lax.fori_loop(0, groups, body, 0)
    plsc.subcore_barrier()
    start = sub * RPT
    pltpu.sync_copy(acc.at[pl.ds(start, RPT)], out_hbm.at[core].at[pl.ds(start, RPT)])


def _dinv_block(dp_r):
    dp = dp_r[...]
    return lax.rsqrt(1.0 + dp[0, :, 0:1] + dp[1, :, 0:1])


def _a_block(bat_r, i):
    gids = lax.broadcasted_iota(jnp.int32, (G, BLK), 0)
    colid = lax.broadcasted_iota(jnp.int32, (G, BLK), 1) + i * BLK
    return jnp.where((bat_r[...] == gids) & (colid < N), 1.0, 0.0)


def _k0_body(x_r, w_r, dp_r, bat_r, hproj_r, hs_r, cnt_r):
    i = pl.program_id(0)
    h = jnp.dot(x_r[...], w_r[...], preferred_element_type=jnp.float32,
                precision=_HI)
    dinv = _dinv_block(dp_r)
    hproj_r[...] = h
    hs = h * dinv
    hs_r[...] = jnp.stack([hs[:, :64], hs[:, 64:]], axis=0)
    a = _a_block(bat_r, i)

    @pl.when(i == 0)
    def _():
        cnt_r[...] = jnp.zeros_like(cnt_r)

    cnt_r[...] += jnp.broadcast_to(jnp.sum(a, axis=1, keepdims=True), (G, 128))


def _tc_prep(xp, W0, degp, batp):
    return pl.pallas_call(
        _k0_body,
        grid=(GRID,),
        in_specs=[
            pl.BlockSpec((BLK, 128), lambda i: (i, 0)),
            pl.BlockSpec((128, 128), lambda i: (0, 0)),
            pl.BlockSpec((2, BLK, 16), lambda i: (0, i, 0)),
            pl.BlockSpec((1, BLK), lambda i: (0, i)),
        ],
        out_specs=[
            pl.BlockSpec((BLK, 128), lambda i: (i, 0)),
            pl.BlockSpec((2, BLK, 64), lambda i: (0, i, 0)),
            pl.BlockSpec((G, 128), lambda i: (0, 0)),
        ],
        out_shape=[
            jax.ShapeDtypeStruct((NPAD, 128), jnp.float32),
            jax.ShapeDtypeStruct((2, NPAD, 64), jnp.float32),
            jax.ShapeDtypeStruct((G, 128), jnp.float32),
        ],
    )(xp, W0, degp, batp)


def _phase0(part_r, hproj_r, dp_r, b_r, a, i, m_s, s1_s, s2_s):
    dinv = _dinv_block(dp_r)
    part = part_r[...]
    s = jnp.concatenate([part[0], part[1]], axis=1)
    m = dinv * s + dinv * dinv * hproj_r[...] + b_r[...]
    m_s[pl.ds(i * BLK, BLK), :] = m

    @pl.when(i == 0)
    def _():
        s1_s[...] = jnp.zeros_like(s1_s)
        s2_s[...] = jnp.zeros_like(s2_s)

    s1_s[...] += jnp.dot(a, m, preferred_element_type=jnp.float32,
                         precision=_HI)
    s2_s[...] += jnp.dot(a, m * m, preferred_element_type=jnp.float32,
                         precision=_HI)


def _phase1_norm(cnt_r, gw_r, gb_r, gm_r, a, i, m_s, s1_s, s2_s):
    cnt = jnp.maximum(cnt_r[...], 1.0)
    mean = s1_s[...] / cnt
    msq = s2_s[...] / cnt
    gm = gm_r[...]
    var = msq - mean * mean * gm * (2.0 - gm)
    istd = lax.rsqrt(var + 1e-5)
    dn = (((0,), (0,)), ((), ()))
    mg = lax.dot_general(a, mean * gm, dn, precision=_HI,
                         preferred_element_type=jnp.float32)
    sb = lax.dot_general(a, istd, dn, precision=_HI,
                         preferred_element_type=jnp.float32)
    m = m_s[pl.ds(i * BLK, BLK), :]
    out = (m - mg) * sb * gw_r[...] + gb_r[...]
    return jnp.maximum(out, 0.0)


_PARK = lambda j, i: i * (1 - j)
_PARK1 = lambda j, i: i * j

_CM_IN_SPECS = [
    pl.BlockSpec((2, BLK, 64), lambda j, i: (0, _PARK(j, i), 0)),
    pl.BlockSpec((BLK, 128), lambda j, i: (_PARK(j, i), 0)),
    pl.BlockSpec((2, BLK, 16), lambda j, i: (0, i, 0)),
    pl.BlockSpec((1, 128), lambda j, i: (0, 0)),
    pl.BlockSpec((1, BLK), lambda j, i: (0, i)),
    pl.BlockSpec((G, 128), lambda j, i: (0, 0)),
    pl.BlockSpec((1, 128), lambda j, i: (0, 0)),
    pl.BlockSpec((1, 128), lambda j, i: (0, 0)),
    pl.BlockSpec((1, 128), lambda j, i: (0, 0)),
]

_CM_SCRATCH = [
    pltpu.VMEM((NPAD, 128), jnp.float32),
    pltpu.VMEM((G, 128), jnp.float32),
    pltpu.VMEM((G, 128), jnp.float32),
]


def _cm_body(part_r, hproj_r, dp_r, b_r, bat_r, cnt_r, gw_r, gb_r, gm_r,
             wn_r, hpn_r, hsn_r, gsum_r, m_s, s1_s, s2_s):
    j = pl.program_id(0)
    i = pl.program_id(1)
    a = _a_block(bat_r, i)

    @pl.when(j == 0)
    def _():
        _phase0(part_r, hproj_r, dp_r, b_r, a, i, m_s, s1_s, s2_s)

    @pl.when(j == 1)
    def _():
        h = _phase1_norm(cnt_r, gw_r, gb_r, gm_r, a, i, m_s, s1_s, s2_s)

        @pl.when(i == 0)
        def _():
            gsum_r[...] = jnp.zeros_like(gsum_r)

        gsum_r[...] += jnp.dot(a, h, preferred_element_type=jnp.float32,
                               precision=_HI)
        hpn = jnp.dot(h, wn_r[...], preferred_element_type=jnp.float32,
                      precision=_HI)
        hpn_r[...] = hpn
        hsn = hpn * _dinv_block(dp_r)
        hsn_r[...] = jnp.stack([hsn[:, :64], hsn[:, 64:]], axis=0)


def _tc_layer(part, hproj, degp, b2d, batp, counts, gw, gb, gm, Wn):
    return pl.pallas_call(
        _cm_body,
        grid=(2, GRID),
        in_specs=_CM_IN_SPECS + [
            pl.BlockSpec((128, 128), lambda j, i: (0, 0)),
        ],
        out_specs=[
            pl.BlockSpec((BLK, 128), lambda j, i: (_PARK1(j, i), 0)),
            pl.BlockSpec((2, BLK, 64), lambda j, i: (0, _PARK1(j, i), 0)),
            pl.BlockSpec((G, 128), lambda j, i: (0, 0)),
        ],
        out_shape=[
            jax.ShapeDtypeStruct((NPAD, 128), jnp.float32),
            jax.ShapeDtypeStruct((2, NPAD, 64), jnp.float32),
            jax.ShapeDtypeStruct((G, 128), jnp.float32),
        ],
        scratch_shapes=_CM_SCRATCH,
    )(part, hproj, degp, b2d, batp, counts, gw, gb, gm, Wn)


def _cm_last_body(part_r, hproj_r, dp_r, b_r, bat_r, cnt_r, gw_r, gb_r, gm_r,
                  g1_r, g2_r, w1_r, b1_r, w2_r, b2_r,
                  h_r, z_r, m_s, s1_s, s2_s, g3_s):
    j = pl.program_id(0)
    i = pl.program_id(1)
    a = _a_block(bat_r, i)

    @pl.when(j == 0)
    def _():
        _phase0(part_r, hproj_r, dp_r, b_r, a, i, m_s, s1_s, s2_s)

    @pl.when(j == 1)
    def _():
        h = _phase1_norm(cnt_r, gw_r, gb_r, gm_r, a, i, m_s, s1_s, s2_s)
        h_r[...] = h

        @pl.when(i == 0)
        def _():
            g3_s[...] = jnp.zeros_like(g3_s)

        g3_s[...] += jnp.dot(a, h, preferred_element_type=jnp.float32,
                             precision=_HI)

        @pl.when(i == GRID - 1)
        def _():
            cnt = jnp.maximum(cnt_r[...], 1.0)
            pooled = jnp.concatenate(
                [g1_r[...] / cnt, g2_r[...] / cnt, g3_s[...] / cnt], axis=1)
            z = jnp.dot(pooled, w1_r[...], preferred_element_type=jnp.float32,
                        precision=_HI) + b1_r[...]
            z = jnp.maximum(z, 0.0)
            z = jnp.dot(z, w2_r[...], preferred_element_type=jnp.float32,
                        precision=_HI) + b2_r[...]
            mask = lax.broadcasted_iota(jnp.int32, (G, 128), 1) < C
            z = jnp.where(mask, z, -jnp.inf)
            zmax = jnp.max(z, axis=1, keepdims=True)
            ez = jnp.where(mask, jnp.exp(z - zmax), 0.0)
            lse = jnp.log(jnp.sum(ez, axis=1, keepdims=True)) + zmax
            z_r[...] = jnp.where(mask, z - lse, 0.0)


def _tc_layer_last(part, hproj, degp, b2d, batp, counts, gw, gb, gm,
                   g1, g2, Wd1, bd1, Wd2p, bd2p):
    return pl.pallas_call(
        _cm_last_body,
        grid=(2, GRID),
        in_specs=_CM_IN_SPECS + [
            pl.BlockSpec((G, 128), lambda j, i: (0, 0)),
            pl.BlockSpec((G, 128), lambda j, i: (0, 0)),
            pl.BlockSpec((3 * H, 3 * H), lambda j, i: (0, 0)),
            pl.BlockSpec((1, 3 * H), lambda j, i: (0, 0)),
            pl.BlockSpec((3 * H, 128), lambda j, i: (0, 0)),
            pl.BlockSpec((1, 128), lambda j, i: (0, 0)),
        ],
        out_specs=[
            pl.BlockSpec((BLK, 128), lambda j, i: (_PARK1(j, i), 0)),
            pl.BlockSpec((G, 128), lambda j, i: (0, 0)),
        ],
        out_shape=[
            jax.ShapeDtypeStruct((NPAD, 128), jnp.float32),
            jax.ShapeDtypeStruct((G, 128), jnp.float32),
        ],
        scratch_shapes=_CM_SCRATCH + [pltpu.VMEM((G, 128), jnp.float32)],
    )(part, hproj, degp, b2d, batp, counts, gw, gb, gm,
      g1, g2, Wd1, bd1, Wd2p, bd2p)


def kernel(x, edge_index, batch, W0, b0, gnw0, gnb0, gnm0, W1, b1, gnw1,
           gnb1, gnm1, W2, b2, gnw2, gnb2, gnm2, Wd1, bd1, Wd2, bd2):
    f32 = jnp.float32
    xp = jnp.pad(x, ((0, NPAD - N), (0, 0)))
    pad_ids = N + jnp.arange(EPAD - E, dtype=jnp.int32) % (NPAD - N)
    srcw = jnp.concatenate(
        [edge_index[0].astype(jnp.int32), pad_ids]).reshape(NW, CH, K)
    dstw = jnp.concatenate(
        [edge_index[1].astype(jnp.int32), pad_ids]).reshape(NW, CH, K)
    batp = jnp.pad(batch.astype(jnp.int32), (0, NPAD - N)).reshape(1, NPAD)

    zeros64 = jnp.zeros((NPAD, 64), f32)
    zeros16 = jnp.zeros((NPAD, 16), f32)
    ones16 = jnp.ones((K, 16), f32)

    degp = _sc_deg(dstw, ones16, zeros16)
    hproj, hs, counts = _tc_prep(xp, W0, degp, batp)

    gsums = []
    for (b, gw, gb, gm, Wn) in ((b0, gnw0, gnb0, gnm0, W1),
                                (b1, gnw1, gnb1, gnm1, W2)):
        part = _sc_scatter(hs, srcw, dstw, zeros64)
        hproj, hs, g = _tc_layer(part, hproj, degp, b.reshape(1, 128), batp,
                                 counts, gw.reshape(1, 128),
                                 gb.reshape(1, 128), gm.reshape(1, 128), Wn)
        gsums.append(g)

    part = _sc_scatter(hs, srcw, dstw, zeros64)
    Wd2p = jnp.pad(Wd2, ((0, 0), (0, 128 - C)))
    bd2p = jnp.pad(bd2, (0, 128 - C)).reshape(1, 128)
    h_last, zp = _tc_layer_last(
        part, hproj, degp, b2.reshape(1, 128), batp, counts,
        gnw2.reshape(1, 128), gnb2.reshape(1, 128), gnm2.reshape(1, 128),
        gsums[0], gsums[1], Wd1, bd1.reshape(1, 3 * H), Wd2p, bd2p)
    return (h_last[:N], zp[:, :C])
